# Initial kernel scaffold; baseline (speedup 1.0000x reference)
import jax
import jax.numpy as jnp
import numpy as np
from jax.experimental import pallas as pl

N_NODES = 10000
MASK_RATE = 0.5
NEG_SLOPE = 0.2
ALPHA_L = 2.0


def _gat_layer(h, src, dst, W, al, ar, b, res_W, act):
    N = h.shape[0]
    H, D = al.shape
    z = (h @ W).reshape(N, H, D)
    el = jnp.sum(z * al[None, :, :], axis=-1)
    er = jnp.sum(z * ar[None, :, :], axis=-1)
    e = jax.nn.leaky_relu(el[src] + er[dst], NEG_SLOPE)
    e_max = jax.ops.segment_max(e, dst, num_segments=N)
    e_max = jnp.where(jnp.isfinite(e_max), e_max, 0.0)
    ex = jnp.exp(e - jax.lax.stop_gradient(e_max)[dst])
    denom = jax.ops.segment_sum(ex, dst, num_segments=N)
    alpha = ex / (denom[dst] + 1e-9)
    out = jax.ops.segment_sum(z[src] * alpha[:, :, None], dst, num_segments=N)
    out = out + b.reshape(1, H, D)
    if res_W is None:
        res = h.reshape(N, H, D)
    else:
        res = (h @ res_W).reshape(N, H, D)
    out = (out + res).reshape(N, H * D)
    if act:
        out = jnp.maximum(out, 0.0)
    return out


def _sce_loss_pallas(x, y, alpha):
    def body(x_ref, y_ref, o_ref):
        xv = x_ref[...]
        yv = y_ref[...]
        xn = xv / (jnp.sqrt(jnp.sum(xv * xv, axis=-1, keepdims=True)) + 1e-8)
        yn = yv / (jnp.sqrt(jnp.sum(yv * yv, axis=-1, keepdims=True)) + 1e-8)
        c = (1.0 - jnp.sum(xn * yn, axis=-1)) ** alpha
        o_ref[0, 0] = jnp.mean(c)

    return pl.pallas_call(
        body,
        out_shape=jax.ShapeDtypeStruct((1, 1), jnp.float32),
    )(x, y)[0, 0]


def kernel(x, edge_index, enc_mask_token, W0, al0, ar0, b0, res0, W1, al1, ar1, b1, W2, al2, ar2, b2, Wd, ald, ard, bd, resd, W_e2d, b_e2d):
    N = x.shape[0]
    perm = jax.random.permutation(jax.random.key(42), N)
    num_mask = int(MASK_RATE * N)
    mask_nodes = perm[:num_mask]
    xm = x.at[mask_nodes].set(enc_mask_token[0])
    src, dst = edge_index[0], edge_index[1]
    h1 = _gat_layer(xm, src, dst, W0, al0, ar0, b0, res0, True)
    h2 = _gat_layer(h1, src, dst, W1, al1, ar1, b1, None, True)
    h3 = _gat_layer(h2, src, dst, W2, al2, ar2, b2, None, True)
    enc_rep = jnp.concatenate([h1, h2, h3], axis=1)
    rep = enc_rep @ W_e2d + b_e2d
    recon = _gat_layer(rep, src, dst, Wd, ald, ard, bd, resd, False)
    x_init = x[mask_nodes]
    x_rec = recon[mask_nodes]
    return _sce_loss_pallas(x_rec, x_init, ALPHA_L)


# speedup vs baseline: 1.0000x; 1.0000x over previous
import jax
import jax.numpy as jnp
import numpy as np
from jax.experimental import pallas as pl

N_NODES = 10000
MASK_RATE = 0.5
NEG_SLOPE = 0.2
ALPHA_L = 2.0


def _gat_layer(h, src, dst, W, al, ar, b, res_W, act):
    N = h.shape[0]
    H, D = al.shape
    z = (h @ W).reshape(N, H, D)
    el = jnp.sum(z * al[None, :, :], axis=-1)
    er = jnp.sum(z * ar[None, :, :], axis=-1)
    e = jax.nn.leaky_relu(el[src] + er[dst], NEG_SLOPE)
    e_max = jax.ops.segment_max(e, dst, num_segments=N)
    e_max = jnp.where(jnp.isfinite(e_max), e_max, 0.0)
    ex = jnp.exp(e - jax.lax.stop_gradient(e_max)[dst])
    denom = jax.ops.segment_sum(ex, dst, num_segments=N)
    alpha = ex / (denom[dst] + 1e-9)
    out = jax.ops.segment_sum(z[src] * alpha[:, :, None], dst, num_segments=N)
    out = out + b.reshape(1, H, D)
    if res_W is None:
        res = h.reshape(N, H, D)
    else:
        res = (h @ res_W).reshape(N, H, D)
    out = (out + res).reshape(N, H * D)
    if act:
        out = jnp.maximum(out, 0.0)
    return out


def _sce_loss_pallas(x, y, alpha):
    def body(x_ref, y_ref, o_ref):
        xv = x_ref[...]
        yv = y_ref[...]
        xn = xv / (jnp.sqrt(jnp.sum(xv * xv, axis=-1, keepdims=True)) + 1e-8)
        yn = yv / (jnp.sqrt(jnp.sum(yv * yv, axis=-1, keepdims=True)) + 1e-8)
        c = (1.0 - jnp.sum(xn * yn, axis=-1)) ** alpha
        o_ref[...] = jnp.mean(c).reshape(1, 1)

    return pl.pallas_call(
        body,
        out_shape=jax.ShapeDtypeStruct((1, 1), jnp.float32),
    )(x, y)[0, 0]


def kernel(x, edge_index, enc_mask_token, W0, al0, ar0, b0, res0, W1, al1, ar1, b1, W2, al2, ar2, b2, Wd, ald, ard, bd, resd, W_e2d, b_e2d):
    N = x.shape[0]
    perm = jax.random.permutation(jax.random.key(42), N)
    num_mask = int(MASK_RATE * N)
    mask_nodes = perm[:num_mask]
    xm = x.at[mask_nodes].set(enc_mask_token[0])
    src, dst = edge_index[0], edge_index[1]
    h1 = _gat_layer(xm, src, dst, W0, al0, ar0, b0, res0, True)
    h2 = _gat_layer(h1, src, dst, W1, al1, ar1, b1, None, True)
    h3 = _gat_layer(h2, src, dst, W2, al2, ar2, b2, None, True)
    enc_rep = jnp.concatenate([h1, h2, h3], axis=1)
    rep = enc_rep @ W_e2d + b_e2d
    recon = _gat_layer(rep, src, dst, Wd, ald, ard, bd, resd, False)
    x_init = x[mask_nodes]
    x_rec = recon[mask_nodes]
    return _sce_loss_pallas(x_rec, x_init, ALPHA_L)


# speedup vs baseline: 1.0102x; 1.0102x over previous
import functools

import jax
import jax.numpy as jnp
import numpy as np
from jax import lax
from jax.experimental import pallas as pl
from jax.experimental.pallas import tpu as pltpu

_N = 10000
_E = 160000
_DIN = 256
_HID = 512
_NH = 8
_HD = 64
_NEG = 0.2
_NMASK = 5000

_RB = 400
_NBLK = _N // _RB


def _mask_np():
    with jax.ensure_compile_time_eval():
        perm = np.asarray(jax.random.permutation(jax.random.key(42), _N))
    mb = np.zeros((_N, 1), np.float32)
    mb[perm[:_NMASK]] = 1.0
    return mb


_MASKBIT = _mask_np()


def _attn_mat(al, ar):
    H, D = al.shape
    A = jnp.zeros((H * D, 2 * H), jnp.float32)
    for h in range(H):
        A = A.at[h * D:(h + 1) * D, h].set(al[h])
        A = A.at[h * D:(h + 1) * D, H + h].set(ar[h])
    return A


def _k1_body(x_ref, mb_ref, tok_ref, W_ref, R_ref, ALR_ref, z_ref, elr_ref, res_ref):
    mb = mb_ref[...]
    xm = x_ref[...] * (1.0 - mb) + tok_ref[...] * mb
    z = jnp.dot(xm, W_ref[...], preferred_element_type=jnp.float32)
    z_ref[...] = z
    elr_ref[...] = jnp.dot(z, ALR_ref[...], preferred_element_type=jnp.float32)
    res_ref[...] = jnp.dot(xm, R_ref[...], preferred_element_type=jnp.float32)


def _layer0_proj(x, mb, tok, W, R, ALR):
    return pl.pallas_call(
        _k1_body,
        grid=(_NBLK,),
        in_specs=[
            pl.BlockSpec((_RB, _DIN), lambda i: (i, 0)),
            pl.BlockSpec((_RB, 1), lambda i: (i, 0)),
            pl.BlockSpec((1, _DIN), lambda i: (0, 0)),
            pl.BlockSpec((_DIN, _HID), lambda i: (0, 0)),
            pl.BlockSpec((_DIN, _HID), lambda i: (0, 0)),
            pl.BlockSpec((_HID, 2 * _NH), lambda i: (0, 0)),
        ],
        out_specs=[
            pl.BlockSpec((_RB, _HID), lambda i: (i, 0)),
            pl.BlockSpec((_RB, 2 * _NH), lambda i: (i, 0)),
            pl.BlockSpec((_RB, _HID), lambda i: (i, 0)),
        ],
        out_shape=[
            jax.ShapeDtypeStruct((_N, _HID), jnp.float32),
            jax.ShapeDtypeStruct((_N, 2 * _NH), jnp.float32),
            jax.ShapeDtypeStruct((_N, _HID), jnp.float32),
        ],
    )(x, mb, tok, W, R, ALR)


def _k3_body(h_ref, W_ref, ALR_ref, z_ref, elr_ref):
    z = jnp.dot(h_ref[...], W_ref[...], preferred_element_type=jnp.float32)
    z_ref[...] = z
    elr_ref[...] = jnp.dot(z, ALR_ref[...], preferred_element_type=jnp.float32)


def _layer_proj(h, W, ALR):
    return pl.pallas_call(
        _k3_body,
        grid=(_NBLK,),
        in_specs=[
            pl.BlockSpec((_RB, _HID), lambda i: (i, 0)),
            pl.BlockSpec((_HID, _HID), lambda i: (0, 0)),
            pl.BlockSpec((_HID, 2 * _NH), lambda i: (0, 0)),
        ],
        out_specs=[
            pl.BlockSpec((_RB, _HID), lambda i: (i, 0)),
            pl.BlockSpec((_RB, 2 * _NH), lambda i: (i, 0)),
        ],
        out_shape=[
            jax.ShapeDtypeStruct((_N, _HID), jnp.float32),
            jax.ShapeDtypeStruct((_N, 2 * _NH), jnp.float32),
        ],
    )(h, W, ALR)


def _k2_body(agg_ref, res_ref, b_ref, h_ref):
    h_ref[...] = jnp.maximum(agg_ref[...] + res_ref[...] + b_ref[...], 0.0)


def _post_layer(aggr, res, b):
    return pl.pallas_call(
        _k2_body,
        grid=(_NBLK,),
        in_specs=[
            pl.BlockSpec((_RB, _HID), lambda i: (i, 0)),
            pl.BlockSpec((_RB, _HID), lambda i: (i, 0)),
            pl.BlockSpec((1, _HID), lambda i: (0, 0)),
        ],
        out_specs=pl.BlockSpec((_RB, _HID), lambda i: (i, 0)),
        out_shape=jax.ShapeDtypeStruct((_N, _HID), jnp.float32),
    )(aggr, res, b.reshape(1, _HID))


def _k4_body(h1_ref, h2_ref, h3_ref, Wa_ref, Wb_ref, Wc_ref, be_ref,
             Wd_ref, ALRd_ref, Rd_ref, zd_ref, elrd_ref, resd_ref):
    rep = jnp.dot(h1_ref[...], Wa_ref[...], preferred_element_type=jnp.float32)
    rep += jnp.dot(h2_ref[...], Wb_ref[...], preferred_element_type=jnp.float32)
    rep += jnp.dot(h3_ref[...], Wc_ref[...], preferred_element_type=jnp.float32)
    rep += be_ref[...]
    zd = jnp.dot(rep, Wd_ref[...], preferred_element_type=jnp.float32)
    zd_ref[...] = zd
    elrd_ref[...] = jnp.dot(zd, ALRd_ref[...], preferred_element_type=jnp.float32)
    resd_ref[...] = jnp.dot(rep, Rd_ref[...], preferred_element_type=jnp.float32)


def _e2d_dec_proj(h1, h2, h3, Wa, Wb, Wc, be, Wd, ALRd, Rd):
    return pl.pallas_call(
        _k4_body,
        grid=(_NBLK,),
        in_specs=[
            pl.BlockSpec((_RB, _HID), lambda i: (i, 0)),
            pl.BlockSpec((_RB, _HID), lambda i: (i, 0)),
            pl.BlockSpec((_RB, _HID), lambda i: (i, 0)),
            pl.BlockSpec((_HID, _HID), lambda i: (0, 0)),
            pl.BlockSpec((_HID, _HID), lambda i: (0, 0)),
            pl.BlockSpec((_HID, _HID), lambda i: (0, 0)),
            pl.BlockSpec((1, _HID), lambda i: (0, 0)),
            pl.BlockSpec((_HID, _DIN), lambda i: (0, 0)),
            pl.BlockSpec((_DIN, 2), lambda i: (0, 0)),
            pl.BlockSpec((_HID, _DIN), lambda i: (0, 0)),
        ],
        out_specs=[
            pl.BlockSpec((_RB, _DIN), lambda i: (i, 0)),
            pl.BlockSpec((_RB, 2), lambda i: (i, 0)),
            pl.BlockSpec((_RB, _DIN), lambda i: (i, 0)),
        ],
        out_shape=[
            jax.ShapeDtypeStruct((_N, _DIN), jnp.float32),
            jax.ShapeDtypeStruct((_N, 2), jnp.float32),
            jax.ShapeDtypeStruct((_N, _DIN), jnp.float32),
        ],
    )(h1, h2, h3, Wa, Wb, Wc, be.reshape(1, _HID), Wd, ALRd, Rd)


def _k5_body(aggd_ref, resd_ref, bd_ref, x_ref, mb_ref, out_ref):
    i = pl.program_id(0)
    rec = aggd_ref[...] + resd_ref[...] + bd_ref[...]
    xb = x_ref[...]
    rn = jnp.sqrt(jnp.sum(rec * rec, axis=1, keepdims=True)) + 1e-8
    xn = jnp.sqrt(jnp.sum(xb * xb, axis=1, keepdims=True)) + 1e-8
    dot = jnp.sum((rec / rn) * (xb / xn), axis=1, keepdims=True)
    c = (1.0 - dot) ** 2
    blk = jnp.sum(c * mb_ref[...]) / float(_NMASK)

    @pl.when(i == 0)
    def _():
        out_ref[...] = jnp.zeros_like(out_ref)

    out_ref[...] += blk.reshape(1, 1)


def _loss(aggd, resd, bd, x, mb):
    return pl.pallas_call(
        _k5_body,
        grid=(_NBLK,),
        in_specs=[
            pl.BlockSpec((_RB, _DIN), lambda i: (i, 0)),
            pl.BlockSpec((_RB, _DIN), lambda i: (i, 0)),
            pl.BlockSpec((1, _DIN), lambda i: (0, 0)),
            pl.BlockSpec((_RB, _DIN), lambda i: (i, 0)),
            pl.BlockSpec((_RB, 1), lambda i: (i, 0)),
        ],
        out_specs=pl.BlockSpec((1, 1), lambda i: (0, 0)),
        out_shape=jax.ShapeDtypeStruct((1, 1), jnp.float32),
    )(aggd, resd, bd.reshape(1, _DIN), x, mb)[0, 0]


def _gat_sparse(z, el, er, src, dst, H, D):
    e = jax.nn.leaky_relu(el[src] + er[dst], _NEG)
    e_max = jax.ops.segment_max(e, dst, num_segments=_N)
    e_max = jnp.where(jnp.isfinite(e_max), e_max, 0.0)
    ex = jnp.exp(e - e_max[dst])
    denom = jax.ops.segment_sum(ex, dst, num_segments=_N)
    alpha = ex / (denom[dst] + 1e-9)
    zs = z.reshape(_N, H, D)[src]
    out = jax.ops.segment_sum(zs * alpha[:, :, None], dst, num_segments=_N)
    return out.reshape(_N, H * D)


def kernel(x, edge_index, enc_mask_token, W0, al0, ar0, b0, res0, W1, al1, ar1, b1,
           W2, al2, ar2, b2, Wd, ald, ard, bd, resd, W_e2d, b_e2d):
    mb = jnp.asarray(_MASKBIT)
    src, dst = edge_index[0], edge_index[1]

    ALR0 = _attn_mat(al0, ar0)
    ALR1 = _attn_mat(al1, ar1)
    ALR2 = _attn_mat(al2, ar2)
    ALRd = _attn_mat(ald, ard)

    z0, elr0, res_h = _layer0_proj(x, mb, enc_mask_token, W0, res0, ALR0)
    a0 = _gat_sparse(z0, elr0[:, :_NH], elr0[:, _NH:], src, dst, _NH, _HD)
    h1 = _post_layer(a0, res_h, b0)

    z1, elr1 = _layer_proj(h1, W1, ALR1)
    a1 = _gat_sparse(z1, elr1[:, :_NH], elr1[:, _NH:], src, dst, _NH, _HD)
    h2 = _post_layer(a1, h1, b1)

    z2, elr2 = _layer_proj(h2, W2, ALR2)
    a2 = _gat_sparse(z2, elr2[:, :_NH], elr2[:, _NH:], src, dst, _NH, _HD)
    h3 = _post_layer(a2, h2, b2)

    Wa, Wb, Wc = W_e2d[:_HID], W_e2d[_HID:2 * _HID], W_e2d[2 * _HID:]
    zd, elrd, resd_out = _e2d_dec_proj(h1, h2, h3, Wa, Wb, Wc, b_e2d, Wd, ALRd, resd)
    ad = _gat_sparse(zd, elrd[:, :1], elrd[:, 1:], src, dst, 1, _DIN)

    return _loss(ad, resd_out, bd, x, mb)
